# Initial kernel scaffold; baseline (speedup 1.0000x reference)
import jax
import jax.numpy as jnp
from jax.experimental import pallas as pl


def kernel(x, table, gamma, beta):
    raise NotImplementedError("write your pallas kernel here")


# speedup vs baseline: 1.1542x; 1.1542x over previous
import jax
import jax.numpy as jnp
import numpy as np
from jax import lax
from jax.experimental import pallas as pl
from jax.experimental.pallas import tpu as pltpu
from jax.experimental.pallas import tpu_sc as plsc

D = 64
LN_EPS = 1e-5
NC = 2
NS = 16
NW = NC * NS


_GATHER_DNUMS = lax.GatherDimensionNumbers(
    offset_dims=(), collapsed_slice_dims=(0,), start_index_map=(0,))


def _lane_sum(x):
    lane = lax.iota(jnp.int32, 16)
    for s in (1, 2, 4, 8):
        p = (lane ^ s).reshape(16, 1)
        x = x + lax.gather(x, p, _GATHER_DNUMS, (1,),
                           mode=lax.GatherScatterMode.PROMISE_IN_BOUNDS)
    return x


def _ln_rows(rows_v, r, g_regs, b_regs):
    v = [rows_v[r, pl.ds(16 * k, 16)] for k in range(4)]
    s = _lane_sum(v[0] + v[1] + v[2] + v[3])
    ss = _lane_sum(v[0] * v[0] + v[1] * v[1] + v[2] * v[2] + v[3] * v[3])
    mean = s * (1.0 / 64.0)
    var = ss * (1.0 / 64.0) - mean * mean
    x = var + LN_EPS
    i = lax.bitcast_convert_type(x, jnp.int32)
    i = jnp.int32(0x5F3759DF) - lax.shift_right_logical(i, 1)
    y = lax.bitcast_convert_type(i, jnp.float32)
    xh = 0.5 * x
    y = y * (1.5 - xh * y * y)
    y = y * (1.5 - xh * y * y)
    y = y * (1.5 - xh * y * y)
    nb = -mean * y
    for k in range(4):
        rows_v[r, pl.ds(16 * k, 16)] = (v[k] * y + nb) * g_regs[k] + b_regs[k]


def _make_sc_call(n_rows, chunk):
    assert n_rows % (NW * chunk) == 0
    rows_per_w = n_rows // NW
    n_chunks = rows_per_w // chunk
    mesh = plsc.VectorSubcoreMesh(core_axis_name="c", subcore_axis_name="s")

    def body(x_hbm, tab_hbm, g_hbm, b_hbm, out_hbm,
             idx_v, rows_v, g_v, b_v, sem):
        wid = lax.axis_index("s") * NC + lax.axis_index("c")
        base = wid * rows_per_w
        pltpu.sync_copy(g_hbm, g_v)
        pltpu.sync_copy(b_hbm, b_v)
        g_regs = [g_v[pl.ds(16 * k, 16)] for k in range(4)]
        b_regs = [b_v[pl.ds(16 * k, 16)] for k in range(4)]

        @pl.loop(0, n_chunks)
        def _chunk(c):
            rbase = base + c * chunk
            pltpu.sync_copy(x_hbm.at[pl.ds(rbase, chunk)], idx_v)
            pltpu.async_copy(tab_hbm.at[idx_v], rows_v, sem).wait()

            @pl.loop(0, chunk)
            def _row(r):
                _ln_rows(rows_v, r, g_regs, b_regs)

            pltpu.sync_copy(rows_v, out_hbm.at[pl.ds(rbase, chunk)])

    return pl.kernel(
        body,
        out_type=jax.ShapeDtypeStruct((n_rows, D), jnp.float32),
        mesh=mesh,
        scratch_types=[
            pltpu.VMEM((chunk,), jnp.int32),
            pltpu.VMEM((chunk, D), jnp.float32),
            pltpu.VMEM((D,), jnp.float32),
            pltpu.VMEM((D,), jnp.float32),
            pltpu.SemaphoreType.DMA,
        ],
        compiler_params=pltpu.CompilerParams(use_tc_tiling_on_sc=False),
    )


def kernel(x, table, gamma, beta):
    b, l = x.shape
    xf = x.reshape(-1)
    out = _make_sc_call(b * l, 512)(xf, table, gamma, beta)
    return out.reshape(b, l, D)


# speedup vs baseline: 1.8417x; 1.5956x over previous
import jax
import jax.numpy as jnp
from jax import lax
from jax.experimental import pallas as pl
from jax.experimental.pallas import tpu as pltpu
from jax.experimental.pallas import tpu_sc as plsc

D = 64
LN_EPS = 1e-5
NC = 2
NS = 16
NW = NC * NS

_GATHER_DNUMS = lax.GatherDimensionNumbers(
    offset_dims=(), collapsed_slice_dims=(0,), start_index_map=(0,))


def _lane_sum(x):
    lane = lax.iota(jnp.int32, 16)
    for s in (1, 2, 4, 8):
        p = (lane ^ s).reshape(16, 1)
        x = x + lax.gather(x, p, _GATHER_DNUMS, (1,),
                           mode=lax.GatherScatterMode.PROMISE_IN_BOUNDS)
    return x


def _ln_row(rows_v, r, g_regs, b_regs):
    v = [rows_v[r, pl.ds(16 * k, 16)] for k in range(4)]
    s = _lane_sum(v[0] + v[1] + v[2] + v[3])
    ss = _lane_sum(v[0] * v[0] + (v[1] * v[1] + (v[2] * v[2] + v[3] * v[3])))
    mean = s * (1.0 / 64.0)
    var = ss * (1.0 / 64.0) - mean * mean
    x = var + LN_EPS
    i = lax.bitcast_convert_type(x, jnp.int32)
    i = jnp.int32(0x5F3759DF) - lax.shift_right_logical(i, 1)
    y = lax.bitcast_convert_type(i, jnp.float32)
    xh = 0.5 * x
    y = y * (1.5 - xh * y * y)
    y = y * (1.5 - xh * y * y)
    nb = -mean * y
    for k in range(4):
        rows_v[r, pl.ds(16 * k, 16)] = (v[k] * y + nb) * g_regs[k] + b_regs[k]


def _make_sc_call(n_rows, chunk):
    assert n_rows % (NW * chunk) == 0
    rows_per_w = n_rows // NW
    n_chunks = rows_per_w // chunk
    mesh = plsc.VectorSubcoreMesh(core_axis_name="c", subcore_axis_name="s")

    def body(x_hbm, tab_hbm, g_hbm, b_hbm, out_hbm,
             idx_all, rows0, rows1, g_v, b_v, sg0, sg1, so0, so1):
        wid = lax.axis_index("s") * NC + lax.axis_index("c")
        base = wid * rows_per_w
        pltpu.sync_copy(x_hbm.at[wid], idx_all)
        pltpu.sync_copy(g_hbm, g_v)
        pltpu.sync_copy(b_hbm, b_v)
        g_regs = [g_v[pl.ds(16 * k, 16)] for k in range(4)]
        b_regs = [b_v[pl.ds(16 * k, 16)] for k in range(4)]
        bufs = ((rows0, sg0, so0), (rows1, sg1, so1))

        def fire_gather(c, rows, sg):
            pltpu.async_copy(tab_hbm.at[idx_all.at[c]], rows, sg)

        def wait_gather(c, rows, sg):
            pltpu.make_async_copy(tab_hbm.at[idx_all.at[c]], rows, sg).wait()

        def fire_out(c, rows, so):
            pltpu.async_copy(rows, out_hbm.at[pl.ds(base + c * chunk, chunk)],
                             so)

        def wait_out(c, rows, so):
            pltpu.make_async_copy(
                rows, out_hbm.at[pl.ds(base + c * chunk, chunk)], so).wait()

        def compute(rows):
            @plsc.parallel_loop(0, chunk, unroll=4)
            def _row(r):
                _ln_row(rows, r, g_regs, b_regs)

        def step(c, me, other, first):
            rows_m, sg_m, so_m = me
            rows_o, sg_o, so_o = other
            if not first:
                wait_out(c - 1, rows_o, so_o)

            @pl.when(c + 1 < n_chunks)
            def _():
                fire_gather(c + 1, rows_o, sg_o)

            wait_gather(c, rows_m, sg_m)
            compute(rows_m)
            fire_out(c, rows_m, so_m)

        fire_gather(0, rows0, sg0)
        step(0, bufs[0], bufs[1], first=True)

        @pl.loop(1, n_chunks)
        def _chunk(c):
            @pl.when((c & 1) == 1)
            def _():
                step(c, bufs[1], bufs[0], first=False)

            @pl.when((c & 1) == 0)
            def _():
                step(c, bufs[0], bufs[1], first=False)

        wait_out(n_chunks - 1, *(bufs[(n_chunks - 1) & 1][0],
                                 bufs[(n_chunks - 1) & 1][2]))

    return pl.kernel(
        body,
        out_type=jax.ShapeDtypeStruct((n_rows, D), jnp.float32),
        mesh=mesh,
        scratch_types=[
            pltpu.VMEM((n_chunks, chunk), jnp.int32),
            pltpu.VMEM((chunk, D), jnp.float32),
            pltpu.VMEM((chunk, D), jnp.float32),
            pltpu.VMEM((D,), jnp.float32),
            pltpu.VMEM((D,), jnp.float32),
            pltpu.SemaphoreType.DMA,
            pltpu.SemaphoreType.DMA,
            pltpu.SemaphoreType.DMA,
            pltpu.SemaphoreType.DMA,
        ],
        compiler_params=pltpu.CompilerParams(use_tc_tiling_on_sc=False),
    )


def kernel(x, table, gamma, beta):
    b, l = x.shape
    n_rows = b * l
    chunk = 512
    xr = x.reshape(NW, n_rows // (NW * chunk), chunk)
    out = _make_sc_call(n_rows, chunk)(xr, table, gamma, beta)
    return out.reshape(b, l, D)
